# baseline (speedup 1.0000x reference)
import functools

import jax
import jax.numpy as jnp
from jax import lax
from jax.experimental import pallas as pl
from jax.experimental.pallas import tpu as pltpu
from jax.experimental.pallas import tpu_sc as plsc

_D = 1024
_ROWS = 4096
_BATCH = 4
_NW = 32
_ROWS_PER_W = _ROWS // _NW
_CHUNK = 32
_NCHUNK = _ROWS_PER_W // _CHUNK
_LPR = _D // 16


def _body(x_hbm, sel_hbm, pos_hbm, out_hbm, idx_v, pe_v, xb_v, sem):
    wid = lax.axis_index("s") * 2 + lax.axis_index("c")
    base = wid * _ROWS_PER_W
    for c in range(_NCHUNK):
        row0 = base + c * _CHUNK
        pltpu.sync_copy(sel_hbm.at[pl.ds(row0, _CHUNK)], idx_v)
        pltpu.async_copy(pos_hbm.at[idx_v], pe_v, sem).wait()
        for b in range(_BATCH):
            pltpu.sync_copy(x_hbm.at[b, pl.ds(row0, _CHUNK)], xb_v)

            def add_step(i, carry):
                r = i >> 6
                j = (i & (_LPR - 1)) * 16
                plsc.addupdate(xb_v.at[r, pl.ds(j, 16)],
                               pe_v[r, pl.ds(j, 16)])
                return carry

            lax.fori_loop(0, _CHUNK * _LPR, add_step, 0, unroll=8)
            pltpu.sync_copy(xb_v, out_hbm.at[b, pl.ds(row0, _CHUNK)])


def kernel(x, pos_table, hash_index):
    sel = hash_index[:64, :64].reshape(-1).astype(jnp.int32)
    pos2 = pos_table.reshape(pos_table.shape[1], _D)
    mesh = plsc.VectorSubcoreMesh(core_axis_name="c", subcore_axis_name="s")
    run = functools.partial(
        pl.kernel,
        out_type=jax.ShapeDtypeStruct((_BATCH, _ROWS, _D), jnp.float32),
        mesh=mesh,
        scratch_types=[
            pltpu.VMEM((_CHUNK,), jnp.int32),
            pltpu.VMEM((_CHUNK, _D), jnp.float32),
            pltpu.VMEM((_CHUNK, _D), jnp.float32),
            pltpu.SemaphoreType.DMA,
        ],
    )(_body)
    return run(x, sel, pos2)


# speedup vs baseline: 1.1856x; 1.1856x over previous
import functools

import jax
import jax.numpy as jnp
from jax import lax
from jax.experimental import pallas as pl
from jax.experimental.pallas import tpu as pltpu
from jax.experimental.pallas import tpu_sc as plsc

_D = 1024
_ROWS = 4096
_BATCH = 4
_NW = 32
_ROWS_PER_W = _ROWS // _NW
_CHUNK = 16
_NCHUNK = _ROWS_PER_W // _CHUNK
_ITEMS = _NCHUNK * _BATCH
_LPR = _D // 16


def _body(x_hbm, sel_hbm, pos_hbm, out_hbm, idx_v, pe_v, xb_v,
          pesem_a, pesem_b, xl0, xl1, xl2, xl3, xs0, xs1, xs2, xs3):
    xl = [xl0, xl1, xl2, xl3]
    xs = [xs0, xs1, xs2, xs3]
    pesem = [pesem_a, pesem_b]
    wid = lax.axis_index("s") * 2 + lax.axis_index("c")
    base = wid * _ROWS_PER_W

    def row0(c):
        return base + c * _CHUNK

    def start_pe(c):
        p = c % 2
        pltpu.sync_copy(sel_hbm.at[pl.ds(row0(c), _CHUNK)], idx_v.at[p])
        return pltpu.async_copy(pos_hbm.at[idx_v.at[p]], pe_v.at[p], pesem[p])

    def start_load(k):
        c, b = divmod(k, _BATCH)
        return pltpu.async_copy(
            x_hbm.at[b, pl.ds(row0(c), _CHUNK)], xb_v.at[k % 4], xl[k % 4])

    def start_store(k):
        c, b = divmod(k, _BATCH)
        return pltpu.async_copy(
            xb_v.at[k % 4], out_hbm.at[b, pl.ds(row0(c), _CHUNK)], xs[k % 4])

    pe_h = {0: start_pe(0)}
    ld_h = {k: start_load(k) for k in range(3)}
    st_h = {}
    for c in range(_NCHUNK):
        p = c % 2
        pe_h[c].wait()
        if c + 1 < _NCHUNK:
            pe_h[c + 1] = start_pe(c + 1)
        for b in range(_BATCH):
            k = c * _BATCH + b
            ld_h[k].wait()

            def add_step(i, carry, q=k % 4, p=p):
                r = i >> 6
                j = (i & (_LPR - 1)) * 16
                plsc.addupdate(xb_v.at[q, r, pl.ds(j, 16)],
                               pe_v[p, r, pl.ds(j, 16)])
                return carry

            lax.fori_loop(0, _CHUNK * _LPR, add_step, 0, unroll=8)
            st_h[k] = start_store(k)
            nk = k + 3
            if nk < _ITEMS:
                if nk >= 4:
                    st_h[nk - 4].wait()
                ld_h[nk] = start_load(nk)
    for k in range(_ITEMS - 4, _ITEMS):
        st_h[k].wait()


def kernel(x, pos_table, hash_index):
    sel = hash_index[:64, :64].reshape(-1).astype(jnp.int32)
    pos2 = pos_table.reshape(pos_table.shape[1], _D)
    mesh = plsc.VectorSubcoreMesh(core_axis_name="c", subcore_axis_name="s")
    run = functools.partial(
        pl.kernel,
        out_type=jax.ShapeDtypeStruct((_BATCH, _ROWS, _D), jnp.float32),
        mesh=mesh,
        scratch_types=[
            pltpu.VMEM((2, _CHUNK), jnp.int32),
            pltpu.VMEM((2, _CHUNK, _D), jnp.float32),
            pltpu.VMEM((4, _CHUNK, _D), jnp.float32),
        ] + [pltpu.SemaphoreType.DMA] * 10,
    )(_body)
    return run(x, sel, pos2)
